# baseline (speedup 1.0000x reference)
import functools

import jax
import jax.numpy as jnp
from jax import lax
from jax.experimental import pallas as pl
from jax.experimental.pallas import tpu as pltpu
from jax.experimental.pallas import tpu_sc as plsc

NC = 2
NS = 16
NW = NC * NS
CHUNK = 128
DEG_W = 16
NBUF = 4


def _mesh():
    return plsc.VectorSubcoreMesh(
        core_axis_name="c", subcore_axis_name="s", num_cores=NC, num_subcores=NS
    )


_SC_PARAMS = pltpu.CompilerParams(use_tc_tiling_on_sc=False)


def _edge_pipeline(gather_slice, src_v, dst_v, acc_sh, rows_v, sems, K):
    gsem = sems[:NBUF]
    ssem = sems[NBUF:]
    T = K // NBUF

    def rslot(b):
        return rows_v.at[pl.ds(b * CHUNK, CHUNK)]

    for b in range(NBUF):
        pltpu.async_copy(gather_slice(src_v.at[b]), rslot(b), gsem[b])

    def outer(t, carry):
        base = t * NBUF
        for b in range(NBUF):
            pltpu.make_async_copy(
                gather_slice(src_v.at[base + b]), rslot(b), gsem[b]).wait()
            pltpu.async_copy(
                rslot(b), acc_sh.at[dst_v.at[base + b]], ssem[b], add=True)

        @pl.when(t + 1 < T)
        def _refill():
            for b in range(NBUF):
                pltpu.make_async_copy(
                    rslot(b), acc_sh.at[dst_v.at[base + b]], ssem[b]).wait()
                pltpu.async_copy(
                    gather_slice(src_v.at[base + NBUF + b]), rslot(b),
                    gsem[b])

        return carry

    lax.fori_loop(0, T, outer, 0)
    for b in range(NBUF):
        pltpu.make_async_copy(
            rslot(b), acc_sh.at[dst_v.at[b]], ssem[b]).wait()


@functools.lru_cache(maxsize=None)
def _agg_colsplit_kernel(DH, K, n_acc):
    rpt = n_acc // NS

    @functools.partial(
        pl.kernel,
        out_type=jax.ShapeDtypeStruct((NC, n_acc, DH), jnp.float32),
        mesh=_mesh(),
        compiler_params=_SC_PARAMS,
        scratch_types=[
            pltpu.VMEM((K, CHUNK), jnp.int32),
            pltpu.VMEM((K, CHUNK), jnp.int32),
            pltpu.VMEM((NBUF * CHUNK, DH), jnp.float32),
            pltpu.VMEM_SHARED((n_acc, DH), jnp.float32),
        ] + [pltpu.SemaphoreType.DMA] * (2 * NBUF),
    )
    def agg(g_hbm, src_hbm, dst_hbm, zeros_hbm, out_hbm,
            src_v, dst_v, rows_v, acc_sh, *sems):
        c = lax.axis_index("c")
        s = lax.axis_index("s")
        row0 = s * rpt
        pltpu.sync_copy(zeros_hbm, acc_sh.at[pl.ds(row0, rpt)])
        pltpu.sync_copy(src_hbm.at[s], src_v)
        pltpu.sync_copy(dst_hbm.at[s], dst_v)
        plsc.subcore_barrier()
        _edge_pipeline(lambda idx: g_hbm.at[c].at[idx],
                       src_v, dst_v, acc_sh, rows_v, sems, K)
        plsc.subcore_barrier()
        pltpu.sync_copy(acc_sh.at[pl.ds(row0, rpt)],
                        out_hbm.at[c, pl.ds(row0, rpt)])

    return agg


@functools.lru_cache(maxsize=None)
def _agg_edgesplit_kernel(D, K, n_acc):
    rpt = n_acc // NS

    @functools.partial(
        pl.kernel,
        out_type=jax.ShapeDtypeStruct((NC, n_acc, D), jnp.float32),
        mesh=_mesh(),
        compiler_params=_SC_PARAMS,
        scratch_types=[
            pltpu.VMEM((K, CHUNK), jnp.int32),
            pltpu.VMEM((K, CHUNK), jnp.int32),
            pltpu.VMEM((NBUF * CHUNK, D), jnp.float32),
            pltpu.VMEM_SHARED((n_acc, D), jnp.float32),
        ] + [pltpu.SemaphoreType.DMA] * (2 * NBUF),
    )
    def agg(g_hbm, src_hbm, dst_hbm, zeros_hbm, out_hbm,
            src_v, dst_v, rows_v, acc_sh, *sems):
        c = lax.axis_index("c")
        s = lax.axis_index("s")
        wid = c * NS + s
        row0 = s * rpt
        pltpu.sync_copy(zeros_hbm, acc_sh.at[pl.ds(row0, rpt)])
        pltpu.sync_copy(src_hbm.at[wid], src_v)
        pltpu.sync_copy(dst_hbm.at[wid], dst_v)
        plsc.subcore_barrier()
        _edge_pipeline(lambda idx: g_hbm.at[idx],
                       src_v, dst_v, acc_sh, rows_v, sems, K)
        plsc.subcore_barrier()
        pltpu.sync_copy(acc_sh.at[pl.ds(row0, rpt)],
                        out_hbm.at[c, pl.ds(row0, rpt)])

    return agg


@functools.lru_cache(maxsize=None)
def _deg_kernel(K, n_acc):
    rpt = n_acc // NS

    @functools.partial(
        pl.kernel,
        out_type=jax.ShapeDtypeStruct((NC, n_acc, DEG_W), jnp.float32),
        mesh=_mesh(),
        compiler_params=_SC_PARAMS,
        scratch_types=[
            pltpu.VMEM((K, CHUNK), jnp.int32),
            pltpu.VMEM((CHUNK, DEG_W), jnp.float32),
            pltpu.VMEM_SHARED((n_acc, DEG_W), jnp.float32),
        ],
    )
    def deg(dst_hbm, ones_hbm, zeros_hbm, out_hbm, dst_v, ones_v, acc_sh):
        c = lax.axis_index("c")
        s = lax.axis_index("s")
        wid = c * NS + s
        row0 = s * rpt
        pltpu.sync_copy(zeros_hbm, acc_sh.at[pl.ds(row0, rpt)])
        pltpu.sync_copy(ones_hbm, ones_v)
        pltpu.sync_copy(dst_hbm.at[wid], dst_v)
        plsc.subcore_barrier()

        def body(j, carry):
            pltpu.sync_copy(ones_v, acc_sh.at[dst_v.at[j]], add=True)
            return carry

        lax.fori_loop(0, K, body, 0)
        plsc.subcore_barrier()
        pltpu.sync_copy(acc_sh.at[pl.ds(row0, rpt)],
                        out_hbm.at[c, pl.ds(row0, rpt)])

    return deg


def _dinv(d0_ref, d1_ref):
    return lax.rsqrt(d0_ref[:, :1] + d1_ref[:, :1] + 1.0)


def _mm_body(x_ref, w_ref, o_ref):
    o_ref[...] = jnp.dot(x_ref[...], w_ref[...],
                         preferred_element_type=jnp.float32)


def _mmscale_body(x_ref, w_ref, d0_ref, d1_ref, o_ref):
    h = jnp.dot(x_ref[...], w_ref[0], preferred_element_type=jnp.float32)
    o_ref[...] = (h * _dinv(d0_ref, d1_ref))[None]


def _mmscale_halves(x, W1h, d0, d1, n, dh):
    bm = _row_block(n)
    return pl.pallas_call(
        _mmscale_body,
        grid=(NC, n // bm),
        in_specs=[
            pl.BlockSpec((bm, x.shape[1]), lambda c, i: (i, 0)),
            pl.BlockSpec((1, x.shape[1], dh), lambda c, i: (c, 0, 0)),
            pl.BlockSpec((bm, DEG_W), lambda c, i: (i, 0)),
            pl.BlockSpec((bm, DEG_W), lambda c, i: (i, 0)),
        ],
        out_specs=pl.BlockSpec((1, bm, dh), lambda c, i: (c, i, 0)),
        out_shape=jax.ShapeDtypeStruct((NC, n, dh), jnp.float32),
    )(x, W1h, d0, d1)


def _scale_body(h_ref, d0_ref, d1_ref, o_ref):
    o_ref[...] = h_ref[...] * _dinv(d0_ref, d1_ref)


def _comb_body(plo_ref, phi_ref, glo_ref, ghi_ref, d0_ref, d1_ref, b_ref,
               w_ref, o_ref):
    dinv = _dinv(d0_ref, d1_ref)
    agg = jnp.concatenate(
        [plo_ref[0] + glo_ref[0], phi_ref[0] + ghi_ref[0]], axis=1)
    h = jnp.maximum(dinv * agg + b_ref[...], 0.0)
    o_ref[...] = jnp.dot(h, w_ref[...],
                         preferred_element_type=jnp.float32) * dinv


def _final_body(q0_ref, q1_ref, g_ref, d0_ref, d1_ref, b_ref, wl_ref, bl_ref,
                o_ref):
    dinv = _dinv(d0_ref, d1_ref)
    h = jnp.maximum(
        dinv * (q0_ref[...] + q1_ref[...] + g_ref[...]) + b_ref[...], 0.0)
    lg = jnp.sum(h * wl_ref[...], axis=1, keepdims=True) + bl_ref[0, 0]
    o_ref[...] = jnp.concatenate([-lg, lg], axis=1)


def _row_block(n):
    return 1000 if n % 1000 == 0 else n


def _tc_call(body, n, out_d, ins):
    bm = _row_block(n)

    def _full_spec(shape):
        nd = len(shape)
        return pl.BlockSpec(shape, lambda i, _nd=nd: (0,) * _nd)

    def _row_spec(shape):
        nd = len(shape)
        return pl.BlockSpec((bm,) + shape[1:],
                            lambda i, _nd=nd: (i,) + (0,) * (_nd - 1))

    specs = [(_row_spec(a.shape) if a.shape[0] == n else _full_spec(a.shape))
             for a in ins]
    return pl.pallas_call(
        body,
        grid=(n // bm,),
        in_specs=specs,
        out_specs=pl.BlockSpec((bm, out_d), lambda i: (i, 0)),
        out_shape=jax.ShapeDtypeStruct((n, out_d), jnp.float32),
    )(*ins)


def _comb(plo, phi, glo, ghi, d0, d1, b1r, W2, n, d2_dim):
    bm = _row_block(n)
    dh = plo.shape[2]
    return pl.pallas_call(
        _comb_body,
        grid=(n // bm,),
        in_specs=[
            pl.BlockSpec((1, bm, dh), lambda i: (0, i, 0)),
            pl.BlockSpec((1, bm, dh), lambda i: (0, i, 0)),
            pl.BlockSpec((1, bm, dh), lambda i: (0, i, 0)),
            pl.BlockSpec((1, bm, dh), lambda i: (0, i, 0)),
            pl.BlockSpec((bm, DEG_W), lambda i: (i, 0)),
            pl.BlockSpec((bm, DEG_W), lambda i: (i, 0)),
            pl.BlockSpec(b1r.shape, lambda i: (0, 0)),
            pl.BlockSpec(W2.shape, lambda i: (0, 0)),
        ],
        out_specs=pl.BlockSpec((bm, d2_dim), lambda i: (i, 0)),
        out_shape=jax.ShapeDtypeStruct((n, d2_dim), jnp.float32),
    )(plo, phi, glo, ghi, d0, d1, b1r, W2)


def kernel(x, edge_index, W1, b1, W2, b2, Wl, bl):
    N = x.shape[0]
    E = edge_index.shape[1]
    d1_dim = W1.shape[1]
    d2_dim = W2.shape[1]
    dh = d1_dim // NC

    def pad_edges(parts):
        ept = parts * CHUNK
        k = -(-(-(-E // ept)) // NBUF) * NBUF
        epad = k * ept
        src = edge_index[0]
        dst = edge_index[1]
        if epad != E:
            src = jnp.concatenate(
                [src, jnp.zeros((epad - E,), edge_index.dtype)])
            dst = jnp.concatenate(
                [dst, jnp.full((epad - E,), N, edge_index.dtype)])
        return src.reshape(parts, k, CHUNK), dst.reshape(parts, k, CHUNK), k

    src1, dst1, K1 = pad_edges(NS)
    src2, dst2, K2 = pad_edges(NW)
    n_acc = -(-(N + 1) // (NS * CHUNK)) * (NS * CHUNK)
    rpt = n_acc // NS
    zeros_deg = jnp.zeros((rpt, DEG_W), jnp.float32)
    ones_deg = jnp.ones((CHUNK, DEG_W), jnp.float32)
    zeros_h = jnp.zeros((rpt, dh), jnp.float32)
    zeros_2 = jnp.zeros((rpt, d2_dim), jnp.float32)

    degp = _deg_kernel(K2, n_acc)(dst2, ones_deg, zeros_deg)
    d0 = degp[0, :N]
    d1 = degp[1, :N]

    w1h = jnp.stack([W1[:, :dh], W1[:, dh:]], axis=0)
    g3 = _mmscale_halves(x, w1h, d0, d1, N, dh)
    p = _agg_colsplit_kernel(dh, K1, n_acc)(g3, src1, dst1, zeros_h)
    g2 = _comb(p[:, :N][0:1], p[:, :N][1:2], g3[0:1], g3[1:2],
               d0, d1, b1.reshape(1, -1), W2, N, d2_dim)

    q = _agg_edgesplit_kernel(d2_dim, K2, n_acc)(g2, src2, dst2, zeros_2)
    out = _tc_call(_final_body, N, 2,
                   [q[0, :N], q[1, :N], g2, d0, d1,
                    b2.reshape(1, -1), Wl.reshape(1, -1),
                    bl.reshape(1, 1)])
    return out


# speedup vs baseline: 1.1244x; 1.1244x over previous
import functools

import jax
import jax.numpy as jnp
from jax import lax
from jax.experimental import pallas as pl
from jax.experimental.pallas import tpu as pltpu
from jax.experimental.pallas import tpu_sc as plsc

NC = 2
NS = 16
NW = NC * NS
CHUNK = 128
DEG_W = 16
NBUF = 4


def _mesh():
    return plsc.VectorSubcoreMesh(
        core_axis_name="c", subcore_axis_name="s", num_cores=NC, num_subcores=NS
    )


_SC_PARAMS = pltpu.CompilerParams(use_tc_tiling_on_sc=False)


def _edge_pipeline(gather_slice, src_v, dst_v, acc_sh, rows_v, sems, K):
    gsem = sems[:NBUF]
    ssem = sems[NBUF:]
    T = K // NBUF

    def rslot(b):
        return rows_v.at[pl.ds(b * CHUNK, CHUNK)]

    for b in range(NBUF):
        pltpu.async_copy(gather_slice(src_v.at[b]), rslot(b), gsem[b])

    def outer(t, carry):
        base = t * NBUF
        for b in range(NBUF):
            pltpu.make_async_copy(
                gather_slice(src_v.at[base + b]), rslot(b), gsem[b]).wait()
            pltpu.async_copy(
                rslot(b), acc_sh.at[dst_v.at[base + b]], ssem[b], add=True)

        @pl.when(t + 1 < T)
        def _refill():
            for b in range(NBUF):
                pltpu.make_async_copy(
                    rslot(b), acc_sh.at[dst_v.at[base + b]], ssem[b]).wait()
                pltpu.async_copy(
                    gather_slice(src_v.at[base + NBUF + b]), rslot(b),
                    gsem[b])

        return carry

    lax.fori_loop(0, T, outer, 0)
    for b in range(NBUF):
        pltpu.make_async_copy(
            rslot(b), acc_sh.at[dst_v.at[b]], ssem[b]).wait()


@functools.lru_cache(maxsize=None)
def _agg_colsplit_kernel(DH, K, n_acc):
    rpt = n_acc // NS

    @functools.partial(
        pl.kernel,
        out_type=jax.ShapeDtypeStruct((NC, n_acc, DH), jnp.float32),
        mesh=_mesh(),
        compiler_params=_SC_PARAMS,
        scratch_types=[
            pltpu.VMEM((K, CHUNK), jnp.int32),
            pltpu.VMEM((K, CHUNK), jnp.int32),
            pltpu.VMEM((NBUF * CHUNK, DH), jnp.float32),
            pltpu.VMEM_SHARED((n_acc, DH), jnp.float32),
        ] + [pltpu.SemaphoreType.DMA] * (2 * NBUF),
    )
    def agg(g_hbm, src_hbm, dst_hbm, zeros_hbm, out_hbm,
            src_v, dst_v, rows_v, acc_sh, *sems):
        c = lax.axis_index("c")
        s = lax.axis_index("s")
        row0 = s * rpt
        pltpu.sync_copy(zeros_hbm, acc_sh.at[pl.ds(row0, rpt)])
        pltpu.sync_copy(src_hbm.at[s], src_v)
        pltpu.sync_copy(dst_hbm.at[s], dst_v)
        plsc.subcore_barrier()
        _edge_pipeline(lambda idx: g_hbm.at[c].at[idx],
                       src_v, dst_v, acc_sh, rows_v, sems, K)
        plsc.subcore_barrier()
        pltpu.sync_copy(acc_sh.at[pl.ds(row0, rpt)],
                        out_hbm.at[c, pl.ds(row0, rpt)])

    return agg


@functools.lru_cache(maxsize=None)
def _agg_edgesplit_kernel(D, K, n_acc):
    rpt = n_acc // NS

    @functools.partial(
        pl.kernel,
        out_type=jax.ShapeDtypeStruct((NC, n_acc, D), jnp.float32),
        mesh=_mesh(),
        compiler_params=_SC_PARAMS,
        scratch_types=[
            pltpu.VMEM((K, CHUNK), jnp.int32),
            pltpu.VMEM((K, CHUNK), jnp.int32),
            pltpu.VMEM((NBUF * CHUNK, D), jnp.float32),
            pltpu.VMEM_SHARED((n_acc, D), jnp.float32),
        ] + [pltpu.SemaphoreType.DMA] * (2 * NBUF),
    )
    def agg(g_hbm, src_hbm, dst_hbm, zeros_hbm, out_hbm,
            src_v, dst_v, rows_v, acc_sh, *sems):
        c = lax.axis_index("c")
        s = lax.axis_index("s")
        wid = c * NS + s
        row0 = s * rpt
        pltpu.sync_copy(zeros_hbm, acc_sh.at[pl.ds(row0, rpt)])
        pltpu.sync_copy(src_hbm.at[wid], src_v)
        pltpu.sync_copy(dst_hbm.at[wid], dst_v)
        plsc.subcore_barrier()
        _edge_pipeline(lambda idx: g_hbm.at[idx],
                       src_v, dst_v, acc_sh, rows_v, sems, K)
        plsc.subcore_barrier()
        pltpu.sync_copy(acc_sh.at[pl.ds(row0, rpt)],
                        out_hbm.at[c, pl.ds(row0, rpt)])

    return agg


@functools.lru_cache(maxsize=None)
def _deg_kernel(K, n_acc):
    rpt = n_acc // NS

    @functools.partial(
        pl.kernel,
        out_type=jax.ShapeDtypeStruct((NC, n_acc, DEG_W), jnp.float32),
        mesh=_mesh(),
        compiler_params=_SC_PARAMS,
        scratch_types=[
            pltpu.VMEM((K, CHUNK), jnp.int32),
            pltpu.VMEM((CHUNK, DEG_W), jnp.float32),
            pltpu.VMEM_SHARED((n_acc, DEG_W), jnp.float32),
        ],
    )
    def deg(dst_hbm, ones_hbm, zeros_hbm, out_hbm, dst_v, ones_v, acc_sh):
        c = lax.axis_index("c")
        s = lax.axis_index("s")
        wid = c * NS + s
        row0 = s * rpt
        pltpu.sync_copy(zeros_hbm, acc_sh.at[pl.ds(row0, rpt)])
        pltpu.sync_copy(ones_hbm, ones_v)
        pltpu.sync_copy(dst_hbm.at[wid], dst_v)
        plsc.subcore_barrier()

        def body(j, carry):
            pltpu.sync_copy(ones_v, acc_sh.at[dst_v.at[j]], add=True)
            return carry

        lax.fori_loop(0, K, body, 0)
        plsc.subcore_barrier()
        pltpu.sync_copy(acc_sh.at[pl.ds(row0, rpt)],
                        out_hbm.at[c, pl.ds(row0, rpt)])

    return deg


def _dinv(d0_ref, d1_ref):
    return lax.rsqrt(d0_ref[:, :1] + d1_ref[:, :1] + 1.0)


def _mm_body(x_ref, w_ref, o_ref):
    o_ref[...] = jnp.dot(x_ref[...], w_ref[...],
                         preferred_element_type=jnp.float32)


def _scale_body(h_ref, d0_ref, d1_ref, o_ref):
    o_ref[...] = h_ref[...] * _dinv(d0_ref, d1_ref)


def _comb_body(plo_ref, phi_ref, g_ref, d0_ref, d1_ref, b_ref,
               w_ref, o_ref):
    dinv = _dinv(d0_ref, d1_ref)
    agg = jnp.concatenate([plo_ref[0], phi_ref[0]], axis=1) + g_ref[...]
    h = jnp.maximum(dinv * agg + b_ref[...], 0.0)
    o_ref[...] = jnp.dot(h, w_ref[...],
                         preferred_element_type=jnp.float32) * dinv


def _final_body(q0_ref, q1_ref, g_ref, d0_ref, d1_ref, b_ref, wl_ref, bl_ref,
                o_ref):
    dinv = _dinv(d0_ref, d1_ref)
    h = jnp.maximum(
        dinv * (q0_ref[...] + q1_ref[...] + g_ref[...]) + b_ref[...], 0.0)
    lg = jnp.sum(h * wl_ref[...], axis=1, keepdims=True) + bl_ref[0, 0]
    o_ref[...] = jnp.concatenate([-lg, lg], axis=1)


def _row_block(n):
    return 1000 if n % 1000 == 0 else n


def _tc_call(body, n, out_d, ins):
    bm = _row_block(n)

    def _full_spec(shape):
        nd = len(shape)
        return pl.BlockSpec(shape, lambda i, _nd=nd: (0,) * _nd)

    def _row_spec(shape):
        nd = len(shape)
        return pl.BlockSpec((bm,) + shape[1:],
                            lambda i, _nd=nd: (i,) + (0,) * (_nd - 1))

    specs = [(_row_spec(a.shape) if a.shape[0] == n else _full_spec(a.shape))
             for a in ins]
    return pl.pallas_call(
        body,
        grid=(n // bm,),
        in_specs=specs,
        out_specs=pl.BlockSpec((bm, out_d), lambda i: (i, 0)),
        out_shape=jax.ShapeDtypeStruct((n, out_d), jnp.float32),
    )(*ins)


def _comb(plo, phi, g1, d0, d1, b1r, W2, n, d2_dim):
    bm = _row_block(n)
    dh = plo.shape[2]
    return pl.pallas_call(
        _comb_body,
        grid=(n // bm,),
        in_specs=[
            pl.BlockSpec((1, bm, dh), lambda i: (0, i, 0)),
            pl.BlockSpec((1, bm, dh), lambda i: (0, i, 0)),
            pl.BlockSpec((bm, g1.shape[1]), lambda i: (i, 0)),
            pl.BlockSpec((bm, DEG_W), lambda i: (i, 0)),
            pl.BlockSpec((bm, DEG_W), lambda i: (i, 0)),
            pl.BlockSpec(b1r.shape, lambda i: (0, 0)),
            pl.BlockSpec(W2.shape, lambda i: (0, 0)),
        ],
        out_specs=pl.BlockSpec((bm, d2_dim), lambda i: (i, 0)),
        out_shape=jax.ShapeDtypeStruct((n, d2_dim), jnp.float32),
    )(plo, phi, g1, d0, d1, b1r, W2)


def kernel(x, edge_index, W1, b1, W2, b2, Wl, bl):
    N = x.shape[0]
    E = edge_index.shape[1]
    d1_dim = W1.shape[1]
    d2_dim = W2.shape[1]
    dh = d1_dim // NC

    def pad_edges(parts):
        ept = parts * CHUNK
        k = -(-(-(-E // ept)) // NBUF) * NBUF
        epad = k * ept
        src = edge_index[0]
        dst = edge_index[1]
        if epad != E:
            src = jnp.concatenate(
                [src, jnp.zeros((epad - E,), edge_index.dtype)])
            dst = jnp.concatenate(
                [dst, jnp.full((epad - E,), N, edge_index.dtype)])
        return src.reshape(parts, k, CHUNK), dst.reshape(parts, k, CHUNK), k

    src1, dst1, K1 = pad_edges(NS)
    src2, dst2, K2 = pad_edges(NW)
    n_acc = -(-(N + 1) // (NS * CHUNK)) * (NS * CHUNK)
    rpt = n_acc // NS
    zeros_deg = jnp.zeros((rpt, DEG_W), jnp.float32)
    ones_deg = jnp.ones((CHUNK, DEG_W), jnp.float32)
    zeros_h = jnp.zeros((rpt, dh), jnp.float32)
    zeros_2 = jnp.zeros((rpt, d2_dim), jnp.float32)

    degp = _deg_kernel(K2, n_acc)(dst2, ones_deg, zeros_deg)
    d0 = degp[0, :N]
    d1 = degp[1, :N]

    h1 = _tc_call(_mm_body, N, d1_dim, [x, W1])
    g1 = _tc_call(_scale_body, N, d1_dim, [h1, d0, d1])

    g3 = jnp.stack([g1[:, :dh], g1[:, dh:]], axis=0)
    p = _agg_colsplit_kernel(dh, K1, n_acc)(g3, src1, dst1, zeros_h)
    g2 = _comb(p[:, :N][0:1], p[:, :N][1:2], g1,
               d0, d1, b1.reshape(1, -1), W2, N, d2_dim)

    q = _agg_edgesplit_kernel(d2_dim, K2, n_acc)(g2, src2, dst2, zeros_2)
    out = _tc_call(_final_body, N, 2,
                   [q[0, :N], q[1, :N], g2, d0, d1,
                    b2.reshape(1, -1), Wl.reshape(1, -1),
                    bl.reshape(1, 1)])
    return out
